# baseline (speedup 1.0000x reference)
import functools

import jax
import jax.numpy as jnp
import numpy as np
from jax.experimental import pallas as pl
from jax.experimental.pallas import tpu as pltpu

EPS = float(np.finfo(np.float32).eps)
NS = 2000
NSP = 2048
NCHUNK = 16
NL = 128
Y0 = 0.9999
STEP = 2.0 * Y0 / (NS - 1)
RSQRT2PI = float(1.0 / np.sqrt(2.0 * np.pi))


def _tables():
    i = np.minimum(np.arange(NSP), NS - 1).astype(np.float64)
    x = (i * STEP - Y0).astype(np.float32)
    t = 0.5 * np.log((1.0 + x) / (1.0 - x) + EPS, dtype=np.float32)
    coef = (RSQRT2PI / (1.0 - x * x)).astype(np.float32)
    lcoef = np.log(coef).astype(np.float32)
    lcoef[NS:] = -np.inf
    return (jnp.asarray(t.astype(np.float32)).reshape(NSP, 1),
            jnp.asarray(lcoef).reshape(NSP, 1))


def _body(t_ref, c_ref, mean_ref, std_ref, u_ref, val_ref, prob_ref,
          tb_ref, lb_ref):
    f32 = jnp.float32
    i32 = jnp.int32
    rb = mean_ref.shape[-1]

    @pl.when(pl.program_id(0) == 0)
    def _bcast_tables():
        tb_ref[...] = jnp.broadcast_to(t_ref[...], (NSP, rb))
        lb_ref[...] = jnp.broadcast_to(c_ref[...], (NSP, rb))

    t = tb_ref[...]
    lcoef = lb_ref[...]
    mean = mean_ref[0]
    std = std_ref[0] + EPS
    u = u_ref[0]
    r = 1.0 / std
    a = -0.5 * r * r

    z = t - mean
    raw = jnp.exp(z * z * a + lcoef) * r

    cs = [jnp.sum(raw[g * NL:(g + 1) * NL, :], axis=0, keepdims=True)
          for g in range(NCHUNK)]
    s = cs[0]
    for g in range(1, NCHUNK):
        s = s + cs[g]
    up = u * (s + EPS)

    acc = jnp.zeros_like(s)
    gst = jnp.zeros(s.shape, i32)
    off = jnp.zeros_like(s)
    for g in range(NCHUNK):
        acc = acc + cs[g]
        m = acc <= up
        gst = gst + m.astype(i32)
        off = off + jnp.where(m, cs[g], 0.0)
    found = gst < NCHUNK
    gs = jnp.minimum(gst, NCHUNK - 1)

    sel = jnp.where(gs == 0, raw[0:NL, :], 0.0)
    for g in range(1, NCHUNK):
        sel = sel + jnp.where(gs == g, raw[g * NL:(g + 1) * NL, :], 0.0)

    ltri = (jax.lax.broadcasted_iota(i32, (NL, NL), 0)
            >= jax.lax.broadcasted_iota(i32, (NL, NL), 1)).astype(f32)
    within = jax.lax.dot_general(ltri, sel, (((1,), (0,)), ((), ())),
                                 preferred_element_type=f32,
                                 precision=jax.lax.Precision.HIGHEST)
    cdfsel = within + off
    cnt = jnp.sum((cdfsel <= up).astype(i32), axis=0, keepdims=True)

    idx = jnp.where(found, gst * NL + cnt, 0)
    idx = jnp.minimum(idx, NS - 1)
    val_ref[0] = idx.astype(f32) * STEP - Y0

    sub = jax.lax.broadcasted_iota(i32, (NL, 1), 0)
    praw = jnp.sum(jnp.where(sub == cnt, sel, 0.0), axis=0, keepdims=True)
    p0 = raw[0:1, :]
    prob_ref[0] = jnp.where(found, praw, p0) / (s + EPS)


@functools.partial(jax.jit, static_argnames=())
def kernel(mean, std, uniform_samples):
    b, a = mean.shape
    rows = b * a
    rb = 256
    nb = rows // rb
    m = mean.reshape(nb, 1, rb)
    s = std.reshape(nb, 1, rb)
    u = uniform_samples.reshape(nb, 1, rb)
    t_tab, c_tab = _tables()
    tab = pl.BlockSpec((NSP, 1), lambda i: (0, 0))
    col = pl.BlockSpec((1, 1, rb), lambda i: (i, 0, 0))
    vals, probs = pl.pallas_call(
        _body,
        grid=(nb,),
        in_specs=[tab, tab, col, col, col],
        out_specs=[col, col],
        out_shape=[
            jax.ShapeDtypeStruct((nb, 1, rb), jnp.float32),
            jax.ShapeDtypeStruct((nb, 1, rb), jnp.float32),
        ],
        scratch_shapes=[
            pltpu.VMEM((NSP, rb), jnp.float32),
            pltpu.VMEM((NSP, rb), jnp.float32),
        ],
    )(t_tab, c_tab, m, s, u)
    return vals.reshape(b, a), probs.reshape(b, a)


# speedup vs baseline: 1.3480x; 1.3480x over previous
import functools

import jax
import jax.numpy as jnp
import numpy as np
from jax.experimental import pallas as pl
from jax.experimental.pallas import tpu as pltpu

EPS = float(np.finfo(np.float32).eps)
NS = 2000
NSP = 2048
NCHUNK = 16
NL = 128
Y0 = 0.9999
STEP = 2.0 * Y0 / (NS - 1)
RSQRT2PI = float(1.0 / np.sqrt(2.0 * np.pi))


def _tables():
    i = np.minimum(np.arange(NSP), NS - 1).astype(np.float64)
    x = (i * STEP - Y0).astype(np.float32)
    t = 0.5 * np.log((1.0 + x) / (1.0 - x) + EPS, dtype=np.float32)
    coef = (RSQRT2PI / (1.0 - x * x)).astype(np.float32)
    lcoef = np.log(coef).astype(np.float32)
    lcoef[NS:] = -np.inf
    return (jnp.asarray(t.astype(np.float32)).reshape(NSP, 1),
            jnp.asarray(lcoef).reshape(NSP, 1))


def _body(t_ref, c_ref, mean_ref, std_ref, u_ref, val_ref, prob_ref):
    f32 = jnp.float32
    i32 = jnp.int32

    t = t_ref[...]
    lcoef = c_ref[...]
    mean = mean_ref[0]
    std = std_ref[0] + EPS
    u = u_ref[0]
    r = 1.0 / std
    a = -0.5 * r * r

    z = t - mean
    raw = jnp.exp(z * z * a + lcoef) * r

    cs = [jnp.sum(raw[g * NL:(g + 1) * NL, :], axis=0, keepdims=True)
          for g in range(NCHUNK)]
    s = cs[0]
    for g in range(1, NCHUNK):
        s = s + cs[g]
    up = u * (s + EPS)

    acc = jnp.zeros_like(s)
    gst = jnp.zeros(s.shape, i32)
    off = jnp.zeros_like(s)
    for g in range(NCHUNK):
        acc = acc + cs[g]
        m = acc <= up
        gst = gst + m.astype(i32)
        off = off + jnp.where(m, cs[g], 0.0)
    found = gst < NCHUNK
    gs = jnp.minimum(gst, NCHUNK - 1)

    sel = jnp.where(gs == 0, raw[0:NL, :], 0.0)
    for g in range(1, NCHUNK):
        sel = sel + jnp.where(gs == g, raw[g * NL:(g + 1) * NL, :], 0.0)

    ltri = (jax.lax.broadcasted_iota(i32, (NL, NL), 0)
            >= jax.lax.broadcasted_iota(i32, (NL, NL), 1)).astype(f32)
    within = jax.lax.dot_general(ltri, sel, (((1,), (0,)), ((), ())),
                                 preferred_element_type=f32,
                                 precision=jax.lax.Precision.HIGHEST)
    cdfsel = within + off
    cnt = jnp.sum((cdfsel <= up).astype(i32), axis=0, keepdims=True)

    idx = jnp.where(found, gst * NL + cnt, 0)
    idx = jnp.minimum(idx, NS - 1)
    val_ref[0] = idx.astype(f32) * STEP - Y0

    sub = jax.lax.broadcasted_iota(i32, (NL, 1), 0)
    praw = jnp.sum(jnp.where(sub == cnt, sel, 0.0), axis=0, keepdims=True)
    p0 = raw[0:1, :]
    prob_ref[0] = jnp.where(found, praw, p0) / (s + EPS)


@functools.partial(jax.jit, static_argnames=())
def kernel(mean, std, uniform_samples):
    b, a = mean.shape
    rows = b * a
    rb = 512
    nb = rows // rb
    m = mean.reshape(nb, 1, rb)
    s = std.reshape(nb, 1, rb)
    u = uniform_samples.reshape(nb, 1, rb)
    t_tab, c_tab = _tables()
    tab = pl.BlockSpec((NSP, 1), lambda i: (0, 0))
    col = pl.BlockSpec((1, 1, rb), lambda i: (i, 0, 0))
    vals, probs = pl.pallas_call(
        _body,
        grid=(nb,),
        in_specs=[tab, tab, col, col, col],
        out_specs=[col, col],
        out_shape=[
            jax.ShapeDtypeStruct((nb, 1, rb), jnp.float32),
            jax.ShapeDtypeStruct((nb, 1, rb), jnp.float32),
        ],
    )(t_tab, c_tab, m, s, u)
    return vals.reshape(b, a), probs.reshape(b, a)


# speedup vs baseline: 1.4657x; 1.0873x over previous
import functools

import jax
import jax.numpy as jnp
import numpy as np
from jax.experimental import pallas as pl
from jax.experimental.pallas import tpu as pltpu

EPS = float(np.finfo(np.float32).eps)
NS = 2000
NSP = 2048
NCHUNK = 16
NL = 128
Y0 = 0.9999
STEP = 2.0 * Y0 / (NS - 1)
RSQRT2PI = float(1.0 / np.sqrt(2.0 * np.pi))


def _tables():
    i = np.minimum(np.arange(NSP), NS - 1).astype(np.float64)
    x = (i * STEP - Y0).astype(np.float32)
    t = 0.5 * np.log((1.0 + x) / (1.0 - x) + EPS, dtype=np.float32)
    coef = (RSQRT2PI / (1.0 - x * x)).astype(np.float32)
    lcoef = np.log(coef).astype(np.float32)
    lcoef[NS:] = -np.inf
    return (jnp.asarray(t.astype(np.float32)).reshape(NSP, 1),
            jnp.asarray(lcoef).reshape(NSP, 1))


def _body(t_ref, c_ref, mean_ref, std_ref, u_ref, val_ref, prob_ref):
    f32 = jnp.float32
    i32 = jnp.int32

    t = t_ref[...]
    lcoef = c_ref[...]
    mean = mean_ref[0]
    std = std_ref[0] + EPS
    u = u_ref[0]
    r = 1.0 / std
    a = -0.5 * r * r

    z = t - mean
    raw = jnp.exp(z * z * a + lcoef) * r

    cs = [jnp.sum(raw[g * NL:(g + 1) * NL, :], axis=0, keepdims=True)
          for g in range(NCHUNK)]
    s = cs[0]
    for g in range(1, NCHUNK):
        s = s + cs[g]
    up = u * (s + EPS)

    acc = jnp.zeros_like(s)
    gst = jnp.zeros(s.shape, i32)
    off = jnp.zeros_like(s)
    for g in range(NCHUNK):
        acc = acc + cs[g]
        m = acc <= up
        gst = gst + m.astype(i32)
        off = off + jnp.where(m, cs[g], 0.0)
    found = gst < NCHUNK
    gs = jnp.minimum(gst, NCHUNK - 1)

    sel = jnp.where(gs == 0, raw[0:NL, :], 0.0)
    for g in range(1, NCHUNK):
        sel = sel + jnp.where(gs == g, raw[g * NL:(g + 1) * NL, :], 0.0)

    ltri = (jax.lax.broadcasted_iota(i32, (NL, NL), 0)
            >= jax.lax.broadcasted_iota(i32, (NL, NL), 1)).astype(f32)
    within = jax.lax.dot_general(ltri, sel, (((1,), (0,)), ((), ())),
                                 preferred_element_type=f32,
                                 precision=jax.lax.Precision.HIGHEST)
    cdfsel = within + off
    cnt = jnp.sum((cdfsel <= up).astype(i32), axis=0, keepdims=True)

    idx = jnp.where(found, gst * NL + cnt, 0)
    idx = jnp.minimum(idx, NS - 1)
    val_ref[0] = idx.astype(f32) * STEP - Y0

    sub = jax.lax.broadcasted_iota(i32, (NL, 1), 0)
    praw = jnp.sum(jnp.where(sub == cnt, sel, 0.0), axis=0, keepdims=True)
    p0 = raw[0:1, :]
    prob_ref[0] = jnp.where(found, praw, p0) / (s + EPS)


@functools.partial(jax.jit, static_argnames=())
def kernel(mean, std, uniform_samples):
    b, a = mean.shape
    rows = b * a
    rb = 1024
    nb = rows // rb
    m = mean.reshape(nb, 1, rb)
    s = std.reshape(nb, 1, rb)
    u = uniform_samples.reshape(nb, 1, rb)
    t_tab, c_tab = _tables()
    tab = pl.BlockSpec((NSP, 1), lambda i: (0, 0))
    col = pl.BlockSpec((1, 1, rb), lambda i: (i, 0, 0))
    vals, probs = pl.pallas_call(
        _body,
        grid=(nb,),
        in_specs=[tab, tab, col, col, col],
        out_specs=[col, col],
        out_shape=[
            jax.ShapeDtypeStruct((nb, 1, rb), jnp.float32),
            jax.ShapeDtypeStruct((nb, 1, rb), jnp.float32),
        ],
    )(t_tab, c_tab, m, s, u)
    return vals.reshape(b, a), probs.reshape(b, a)
